# baseline (speedup 1.0000x reference)
import functools

import jax
import jax.numpy as jnp
from jax import lax
from jax.experimental import pallas as pl
from jax.experimental.pallas import tpu as pltpu
from jax.experimental.pallas import tpu_sc as plsc

F = 26
V = 100000
E = 16
B = 16384
NBUF = 4


def _detile_body(x_ref, o_ref):
    VC = 2048
    nfull, rem = divmod(V, VC)
    row = jax.lax.broadcasted_iota(jnp.int32, (E, 128), 0)
    col = jax.lax.broadcasted_iota(jnp.int32, (E, 128), 1)
    S = [
        (col == 16 * i + row).astype(jnp.float32) for i in range(8)
    ]

    def do_chunk(j, t, vc):
        xs = x_ref[j, :, pl.ds(t * VC, vc)]
        y = jnp.transpose(xs, (1, 0))
        ys = y.reshape(vc // 8, 8, E)
        z = sum(
            jax.lax.dot_general(
                ys[:, i, :], S[i], (((1,), (0,)), ((), ())),
                preferred_element_type=jnp.float32,
            )
            for i in range(8)
        )
        r0 = j * (V // 8) + t * (VC // 8)
        o_ref[pl.ds(r0, vc // 8), :] = z

    def step(k, c):
        do_chunk(k // nfull, k % nfull, VC)
        return c

    lax.fori_loop(0, 2 * nfull, step, 0)
    if rem:
        for j in range(2):
            do_chunk(j, nfull, rem)


def _tc_relayout(tabt):
    return pl.pallas_call(
        _detile_body,
        grid=(F // 2,),
        in_specs=[pl.BlockSpec((2, E, V), lambda i: (i, 0, 0))],
        out_specs=pl.BlockSpec((2 * V * E // 128, 128), lambda i: (i, 0)),
        out_shape=jax.ShapeDtypeStruct((F * V * E // 128, 128), jnp.float32),
        compiler_params=pltpu.CompilerParams(vmem_limit_bytes=60 * 1024 * 1024),
    )(tabt)


def kernel(indices, tables):
    idx32 = indices.astype(jnp.int32)
    tabt = jnp.transpose(tables, (0, 2, 1))
    tab = _tc_relayout(tabt).reshape(F * V, E)

    info = plsc.get_sparse_core_info()
    NC, NS, L = info.num_cores, info.num_subcores, info.num_lanes
    NW = NC * NS
    b_per_w = B // NW

    @functools.partial(
        pl.kernel,
        mesh=plsc.VectorSubcoreMesh(core_axis_name="c", subcore_axis_name="s"),
        out_type=jax.ShapeDtypeStruct((B, F * E), jnp.float32),
        compiler_params=pltpu.CompilerParams(use_tc_tiling_on_sc=False),
        scratch_types=[
            pltpu.VMEM((F, b_per_w), jnp.int32),
            pltpu.VMEM((NBUF, b_per_w, E), jnp.float32),
            pltpu.SemaphoreType.DMA((NBUF,)),
            pltpu.SemaphoreType.DMA((NBUF,)),
        ],
    )
    def k(tab_hbm, idx_hbm, out_hbm, idx_all, rows, gsem, ssem):
        tab_rows = tab_hbm
        wid = lax.axis_index("s") * NC + lax.axis_index("c")
        base = wid * b_per_w

        pltpu.sync_copy(idx_hbm.at[:, pl.ds(base, b_per_w)], idx_all)

        n_vec = b_per_w // L

        def add_off(i, c):
            f = i // n_vec
            j = i - f * n_vec
            sl = pl.ds(j * L, L)
            idx_all[f, sl] = idx_all[f, sl] + f * V
            return c

        lax.fori_loop(0, F * n_vec, add_off, 0)

        def gather(f):
            b = f % NBUF
            return pltpu.async_copy(
                tab_rows.at[idx_all.at[f]], rows.at[b], gsem.at[b]
            )

        def store(f):
            b = f % NBUF
            return pltpu.async_copy(
                rows.at[b],
                out_hbm.at[pl.ds(base, b_per_w), pl.ds(f * E, E)],
                ssem.at[b],
            )

        ghand = {}
        shand = {}
        for f in range(F):
            if f >= NBUF:
                shand[f - NBUF].wait()
            ghand[f] = gather(f)
            if f >= 2:
                ghand[f - 2].wait()
                shand[f - 2] = store(f - 2)
        for f in (F - 2, F - 1):
            ghand[f].wait()
            shand[f] = store(f)
        for f in range(F - NBUF, F):
            shand[f].wait()

    return k(tab, idx32)


# speedup vs baseline: 1.1787x; 1.1787x over previous
import functools

import jax
import jax.numpy as jnp
from jax import lax
from jax.experimental import pallas as pl
from jax.experimental.pallas import tpu as pltpu
from jax.experimental.pallas import tpu_sc as plsc

F = 26
V = 100000
E = 16
B = 16384
NBUF = 4


def _detile_body(x_ref, o_ref):
    VC = 8192
    nfull, rem = divmod(V, VC)
    row = jax.lax.broadcasted_iota(jnp.int32, (E, 128), 0)
    col = jax.lax.broadcasted_iota(jnp.int32, (E, 128), 1)
    S = [
        (col == 16 * i + row).astype(jnp.float32) for i in range(8)
    ]

    def do_chunk(j, t, vc):
        xs = x_ref[j, :, pl.ds(t * VC, vc)]
        y = jnp.transpose(xs, (1, 0))
        ys = y.reshape(vc // 8, 8, E)
        z = sum(
            jax.lax.dot_general(
                ys[:, i, :], S[i], (((1,), (0,)), ((), ())),
                preferred_element_type=jnp.float32,
            )
            for i in range(8)
        )
        r0 = j * (V // 8) + t * (VC // 8)
        o_ref[pl.ds(r0, vc // 8), :] = z

    def step(k, c):
        do_chunk(k // nfull, k % nfull, VC)
        return c

    lax.fori_loop(0, 2 * nfull, step, 0)
    if rem:
        for j in range(2):
            do_chunk(j, nfull, rem)


def _tc_relayout(tabt):
    return pl.pallas_call(
        _detile_body,
        grid=(F // 2,),
        in_specs=[pl.BlockSpec((2, E, V), lambda i: (i, 0, 0))],
        out_specs=pl.BlockSpec((2 * V * E // 128, 128), lambda i: (i, 0)),
        out_shape=jax.ShapeDtypeStruct((F * V * E // 128, 128), jnp.float32),
        compiler_params=pltpu.CompilerParams(vmem_limit_bytes=60 * 1024 * 1024),
    )(tabt)


def kernel(indices, tables):
    idx32 = indices.astype(jnp.int32)
    tabt = jnp.transpose(tables, (0, 2, 1))
    tab = _tc_relayout(tabt).reshape(F * V, E)

    info = plsc.get_sparse_core_info()
    NC, NS, L = info.num_cores, info.num_subcores, info.num_lanes
    NW = NC * NS
    b_per_w = B // NW

    @functools.partial(
        pl.kernel,
        mesh=plsc.VectorSubcoreMesh(core_axis_name="c", subcore_axis_name="s"),
        out_type=jax.ShapeDtypeStruct((B, F * E), jnp.float32),
        compiler_params=pltpu.CompilerParams(use_tc_tiling_on_sc=False),
        scratch_types=[
            pltpu.VMEM((F, b_per_w), jnp.int32),
            pltpu.VMEM((NBUF, b_per_w, E), jnp.float32),
            pltpu.SemaphoreType.DMA((NBUF,)),
            pltpu.SemaphoreType.DMA((NBUF,)),
        ],
    )
    def k(tab_hbm, idx_hbm, out_hbm, idx_all, rows, gsem, ssem):
        tab_rows = tab_hbm
        wid = lax.axis_index("s") * NC + lax.axis_index("c")
        base = wid * b_per_w

        pltpu.sync_copy(idx_hbm.at[:, pl.ds(base, b_per_w)], idx_all)

        n_vec = b_per_w // L

        def add_off(i, c):
            f = i // n_vec
            j = i - f * n_vec
            sl = pl.ds(j * L, L)
            idx_all[f, sl] = idx_all[f, sl] + f * V
            return c

        lax.fori_loop(0, F * n_vec, add_off, 0)

        def gather(f):
            b = f % NBUF
            return pltpu.async_copy(
                tab_rows.at[idx_all.at[f]], rows.at[b], gsem.at[b]
            )

        def store(f):
            b = f % NBUF
            return pltpu.async_copy(
                rows.at[b],
                out_hbm.at[pl.ds(base, b_per_w), pl.ds(f * E, E)],
                ssem.at[b],
            )

        ghand = {}
        shand = {}
        for f in range(F):
            if f >= NBUF:
                shand[f - NBUF].wait()
            ghand[f] = gather(f)
            if f >= 2:
                ghand[f - 2].wait()
                shand[f - 2] = store(f - 2)
        for f in (F - 2, F - 1):
            ghand[f].wait()
            shand[f] = store(f)
        for f in range(F - NBUF, F):
            shand[f].wait()

    return k(tab, idx32)
